# baseline (speedup 1.0000x reference)
import functools

import jax
import jax.numpy as jnp
from jax import lax
from jax.experimental import pallas as pl
from jax.experimental.pallas import tpu as pltpu
from jax.experimental.pallas import tpu_sc as plsc

_L = 200
_D = 64
_LANES = 16
_BLK = 128
_PER_W = 8
_NW_ACT = _L // _PER_W


def _sc_gather_body(x0_hbm, inlen_hbm, tableT_hbm, part_hbm, idx_v, inlen_v,
                    blk_v, acc_v, sem):
    cid = lax.axis_index("c")
    sid = lax.axis_index("s")
    wid = sid * 2 + cid

    @pl.when(wid < _NW_ACT)
    def _():
        base = pl.multiple_of(wid * _PER_W, 8)
        pltpu.sync_copy(x0_hbm.at[pl.ds(base, _LANES)], idx_v)
        pltpu.sync_copy(inlen_hbm.at[pl.ds(0, _LANES)], inlen_v)
        v = idx_v[...]
        n = inlen_v[...][0]
        lane = lax.iota(jnp.int32, _LANES)

        cps = []
        for j in range(_PER_W):
            rb = pl.multiple_of(
                lax.shift_left(lax.shift_right_logical(v[j], 7), 7), _BLK)
            cps.append(
                pltpu.async_copy(tableT_hbm.at[:, pl.ds(rb, _BLK)],
                                 blk_v.at[j], sem))
        for cp in cps:
            cp.wait()

        sels = []
        for j in range(_PER_W):
            p = lax.bitwise_and(v[j], _BLK - 1)
            pg = pl.multiple_of(
                lax.shift_left(lax.shift_right_logical(p, 4), 4), _LANES)
            onehot = lane == lax.bitwise_and(p, _LANES - 1)
            scale = jnp.where(base + j < n, 1.0, 0.0).astype(jnp.float32)
            sels.append((pg, onehot, scale))

        def red(c, carry):
            acc = jnp.zeros((_LANES,), jnp.float32)
            for j, (pg, onehot, scale) in enumerate(sels):
                acc += jnp.where(onehot, blk_v[j, c, pl.ds(pg, _LANES)],
                                 0.0) * scale
            acc_v[c, :] = acc
            return carry

        lax.fori_loop(0, _D, red, 0)

        pltpu.sync_copy(acc_v, part_hbm.at[wid])


def _tc_sum_body(part_vmem, out_vmem):
    out_vmem[0, 0, :] = jnp.sum(part_vmem[...], axis=(0, 2))


def kernel(x_in, in_len, table, requires_grad):
    del requires_grad
    x0 = jnp.pad(x_in[0].astype(jnp.int32), (0, 56))
    in_len = in_len.astype(jnp.int32)
    tableT = table.T
    mesh = plsc.VectorSubcoreMesh(core_axis_name="c", subcore_axis_name="s")
    gather = pl.kernel(
        _sc_gather_body,
        out_type=jax.ShapeDtypeStruct((_NW_ACT, _D, _LANES), jnp.float32),
        mesh=mesh,
        scratch_types=[
            pltpu.VMEM((_LANES,), jnp.int32),
            pltpu.VMEM((_LANES,), jnp.int32),
            pltpu.VMEM((_PER_W, _D, _BLK), jnp.float32),
            pltpu.VMEM((_D, _LANES), jnp.float32),
            pltpu.SemaphoreType.DMA,
        ],
    )
    parts = gather(x0, in_len, tableT)

    return jnp.sum(parts, axis=(0, 2))[None, None, :]


# speedup vs baseline: 1.0010x; 1.0010x over previous
import functools

import jax
import jax.numpy as jnp
from jax import lax
from jax.experimental import pallas as pl
from jax.experimental.pallas import tpu as pltpu
from jax.experimental.pallas import tpu_sc as plsc

_L = 200
_D = 64
_LANES = 16
_BLK = 128
_PER_W = 8
_NW_ACT = _L // _PER_W


def _sc_gather_body(x0_hbm, inlen_hbm, tableT_hbm, part_hbm, idx_v, inlen_v,
                    blk_v, acc_v, sem):
    cid = lax.axis_index("c")
    sid = lax.axis_index("s")
    wid = sid * 2 + cid

    @pl.when(wid < _NW_ACT)
    def _():
        base = pl.multiple_of(wid * _PER_W, 8)
        pltpu.sync_copy(x0_hbm.at[pl.ds(base, _LANES)], idx_v)
        pltpu.sync_copy(inlen_hbm.at[pl.ds(0, _LANES)], inlen_v)
        v = idx_v[...]
        n = inlen_v[...][0]
        lane = lax.iota(jnp.int32, _LANES)

        cps = []
        for j in range(_PER_W):
            rb = pl.multiple_of(
                lax.shift_left(lax.shift_right_logical(v[j], 7), 7), _BLK)
            cps.append(
                pltpu.async_copy(tableT_hbm.at[:, pl.ds(rb, _BLK)],
                                 blk_v.at[j], sem))
        for cp in cps:
            cp.wait()

        sels = []
        for j in range(_PER_W):
            p = lax.bitwise_and(v[j], _BLK - 1)
            pg = pl.multiple_of(
                lax.shift_left(lax.shift_right_logical(p, 4), 4), _LANES)
            onehot = lane == lax.bitwise_and(p, _LANES - 1)
            scale = jnp.where(base + j < n, 1.0, 0.0).astype(jnp.float32)
            sels.append((pg, onehot, scale))

        def red(c, carry):
            acc = jnp.zeros((_LANES,), jnp.float32)
            for j, (pg, onehot, scale) in enumerate(sels):
                acc += jnp.where(onehot, blk_v[j, c, pl.ds(pg, _LANES)],
                                 0.0) * scale
            acc_v[c, :] = acc
            return carry

        lax.fori_loop(0, _D, red, 0)

        pltpu.sync_copy(acc_v, part_hbm.at[wid])


def _tc_sum_body(part_vmem, out_vmem):
    out_vmem[0, 0, :] = jnp.sum(part_vmem[...], axis=(0, 2))


def kernel(x_in, in_len, table, requires_grad):
    del requires_grad
    x0 = jnp.pad(x_in[0].astype(jnp.int32), (0, 56))
    in_len = in_len.astype(jnp.int32)
    tableT = table.T
    mesh = plsc.VectorSubcoreMesh(core_axis_name="c", subcore_axis_name="s")
    gather = pl.kernel(
        _sc_gather_body,
        out_type=jax.ShapeDtypeStruct((_NW_ACT, _D, _LANES), jnp.float32),
        mesh=mesh,
        scratch_types=[
            pltpu.VMEM((_LANES,), jnp.int32),
            pltpu.VMEM((_LANES,), jnp.int32),
            pltpu.VMEM((_PER_W, _D, _BLK), jnp.float32),
            pltpu.VMEM((_D, _LANES), jnp.float32),
            pltpu.SemaphoreType.DMA,
        ],
    )
    parts = gather(x0, in_len, tableT)

    out = pl.pallas_call(
        _tc_sum_body,
        out_shape=jax.ShapeDtypeStruct((1, 1, _D), jnp.float32),
    )(parts)
    return out
